# baseline (speedup 1.0000x reference)
import functools

import jax
import jax.numpy as jnp
from jax import lax
from jax.experimental import pallas as pl
from jax.experimental.pallas import tpu as pltpu
from jax.experimental.pallas import tpu_sc as plsc

_N_SEG = 30
_INV_INTERVAL = float(_N_SEG)

_NW = 32
_LANES = 16
_ROWS = 32
_COLS = 512


def _build_tables(y_v, tab_a, tab_b):
    ks = lax.iota(jnp.int32, _LANES).astype(jnp.float32)
    for base in (0, 14):
        yl = y_v[pl.ds(base, _LANES)]
        yr = y_v[pl.ds(base + 1, _LANES)]
        a = (yr - yl) * _INV_INTERVAL
        b = yl - a * ((ks + float(base)) * (1.0 / _N_SEG))
        tab_a[pl.ds(base, _LANES)] = a
        tab_b[pl.ds(base, _LANES)] = b


def _compute_chunk(tab_a, tab_b, src, dst):
    @plsc.parallel_loop(0, _ROWS * (_COLS // _LANES), unroll=8)
    def _(i):
        r = i >> 5
        c = (i & 31) << 4
        xv = src[r, pl.ds(c, _LANES)]
        idx = (xv * _INV_INTERVAL).astype(jnp.int32)
        idx = jnp.minimum(jnp.maximum(idx, 0), _N_SEG - 1)
        a = plsc.load_gather(tab_a, [idx])
        b = plsc.load_gather(tab_b, [idx])
        y = jnp.minimum(jnp.maximum(xv * a + b, 0.0), 1.0)
        dst[r, pl.ds(c, _LANES)] = y


def _tone_body(n_chunks, x_hbm, y_hbm, out_hbm, y_v, tab_a, tab_b,
               in0, in1, out0, out1, si0, si1, so0, so1):
    wid = lax.axis_index("s") * 2 + lax.axis_index("c")
    base = wid * n_chunks
    pltpu.sync_copy(y_hbm, y_v)
    _build_tables(y_v, tab_a, tab_b)

    ins, outs, sis, sos = (in0, in1), (out0, out1), (si0, si1), (so0, so1)

    def in_slice(ci):
        return x_hbm.at[(base + ci) >> 4, pl.ds(((base + ci) & 15) * _ROWS, _ROWS), :]

    def out_slice(ci):
        return out_hbm.at[(base + ci) >> 4, pl.ds(((base + ci) & 15) * _ROWS, _ROWS), :]

    pltpu.async_copy(in_slice(0), in0, si0)
    pltpu.async_copy(in_slice(1), in1, si1)

    def pair_body(g, _):
        for b in range(2):
            ci = g * 2 + b
            pltpu.make_async_copy(in_slice(ci), ins[b], sis[b]).wait()

            @pl.when(g > 0)
            def _():
                pltpu.make_async_copy(outs[b], out_slice(ci), sos[b]).wait()

            _compute_chunk(tab_a, tab_b, ins[b], outs[b])
            pltpu.async_copy(outs[b], out_slice(ci), sos[b])

            @pl.when(ci + 2 < n_chunks)
            def _():
                pltpu.async_copy(in_slice(ci + 2), ins[b], sis[b])
        return 0

    lax.fori_loop(0, n_chunks // 2, pair_body, 0)
    pltpu.make_async_copy(out0, out_slice(n_chunks - 2), so0).wait()
    pltpu.make_async_copy(out1, out_slice(n_chunks - 1), so1).wait()


@functools.partial(jax.jit, static_argnames=("planes",))
def _tone_map(x3, y_pos, planes):
    n_chunks = planes * (512 // _ROWS) // _NW
    body = functools.partial(_tone_body, n_chunks)
    return pl.kernel(
        body,
        out_type=jax.ShapeDtypeStruct((planes, 512, 512), jnp.float32),
        mesh=plsc.VectorSubcoreMesh(core_axis_name="c", subcore_axis_name="s"),
        compiler_params=pltpu.CompilerParams(
            needs_layout_passes=False, use_tc_tiling_on_sc=True),
        scratch_types=[
            pltpu.VMEM((31,), jnp.float32),
            pltpu.VMEM((32,), jnp.float32),
            pltpu.VMEM((32,), jnp.float32),
            pltpu.VMEM((_ROWS, _COLS), jnp.float32),
            pltpu.VMEM((_ROWS, _COLS), jnp.float32),
            pltpu.VMEM((_ROWS, _COLS), jnp.float32),
            pltpu.VMEM((_ROWS, _COLS), jnp.float32),
            pltpu.SemaphoreType.DMA,
            pltpu.SemaphoreType.DMA,
            pltpu.SemaphoreType.DMA,
            pltpu.SemaphoreType.DMA,
        ],
    )(x3, y_pos)


def kernel(x, y_pos):
    planes = x.shape[0] * x.shape[1]
    x3 = x.reshape((planes, x.shape[2], x.shape[3]))
    out = _tone_map(x3, y_pos, planes)
    return (out.reshape(x.shape),)
